# Initial kernel scaffold; baseline (speedup 1.0000x reference)
import jax
import jax.numpy as jnp
from jax.experimental import pallas as pl


def kernel(x, edge_index, edge_weight, W1, b1, W2, b2):
    raise NotImplementedError("write your pallas kernel here")


# speedup vs baseline: 3.9592x; 3.9592x over previous
import functools

import jax
import jax.numpy as jnp
from jax import lax
from jax.experimental import pallas as pl
from jax.experimental.pallas import tpu as pltpu
from jax.experimental.pallas import tpu_sc as plsc

N = 10000
E = 320000
D = 128
NC, NS, L = 2, 16, 16
NW = NC * NS
N_PAD = 10240
HI = N_PAD // 128
RPT = N_PAD // NS
B = 64
NB = 160
E_PAD = NW * B * NB
_CH = 32
_NCH = RPT // _CH


@functools.lru_cache(maxsize=None)
def _make_sc_agg():
    mesh = plsc.VectorSubcoreMesh(core_axis_name="c", subcore_axis_name="s",
                                  num_cores=NC, num_subcores=NS)
    out_type = jax.ShapeDtypeStruct((NC, N_PAD, D), jnp.float32)
    scratch = [
        pltpu.VMEM_SHARED((N_PAD, D), jnp.float32),
        pltpu.VMEM((B,), jnp.int32),
        pltpu.VMEM((B,), jnp.int32),
        pltpu.VMEM((B,), jnp.float32),
        pltpu.VMEM((B, D), jnp.float32),
        pltpu.VMEM((_CH, D), jnp.float32),
        pltpu.SemaphoreType.DMA,
    ]

    @functools.partial(pl.kernel, out_type=out_type, mesh=mesh,
                       scratch_types=scratch)
    def sc_agg(feat, srcr, dstr, wr, zrows, agg_out,
               aggs, idxv, dstv, wv, rows, tbuf, sem):
        c = lax.axis_index("c")
        s = lax.axis_index("s")
        wid = c * NS + s
        row0 = s * RPT

        pltpu.sync_copy(zrows, tbuf)
        for j in range(_NCH):
            pltpu.sync_copy(tbuf, aggs.at[pl.ds(row0 + j * _CH, _CH)])
        plsc.subcore_barrier()

        base_e = wid * (NB * B)

        def batch_body(bi, _):
            off = base_e + bi * B
            pltpu.sync_copy(srcr.at[pl.ds(off, B)], idxv)
            pltpu.sync_copy(dstr.at[pl.ds(off, B)], dstv)
            pltpu.sync_copy(wr.at[pl.ds(off, B)], wv)
            pltpu.async_copy(feat.at[idxv], rows, sem).wait()
            def g_body(g, _):
                w16 = wv[pl.ds(g * L, L)]
                for k in range(L):
                    wb = jnp.broadcast_to(w16[k], (L,))
                    e = g * L + k
                    for cc in range(D // L):
                        sl = pl.ds(cc * L, L)
                        rows[e, sl] = rows[e, sl] * wb
                return 0
            lax.fori_loop(0, B // L, g_body, 0)
            pltpu.sync_copy(rows, aggs.at[dstv], add=True)
            return 0

        lax.fori_loop(0, NB, batch_body, 0)
        plsc.subcore_barrier()

        for j in range(_NCH):
            pltpu.sync_copy(aggs.at[pl.ds(row0 + j * _CH, _CH)], tbuf)
            pltpu.sync_copy(tbuf, agg_out.at[c, pl.ds(row0 + j * _CH, _CH)])

    return sc_agg


_ER = 512
_EBR = 8


def _deg_body(dst_ref, deg_ref):
    @pl.when(pl.program_id(0) == 0)
    def _():
        deg_ref[...] = jnp.zeros_like(deg_ref)
    acc = deg_ref[...]
    lanes = lax.broadcasted_iota(jnp.int32, (1, 128), 1)
    his = lax.broadcasted_iota(jnp.int32, (HI, 1), 0)
    for r in range(_EBR):
        d = dst_ref[r, :]
        oh_lo = (d[:, None] % 128 == lanes).astype(jnp.float32)
        oh_hi = (d[None, :] // 128 == his).astype(jnp.float32)
        acc += jnp.dot(oh_hi, oh_lo, preferred_element_type=jnp.float32)
    deg_ref[...] = acc


def _tc_degree(dst2d):
    return pl.pallas_call(
        _deg_body,
        grid=(E_PAD // (_ER * _EBR),),
        in_specs=[pl.BlockSpec((_EBR, _ER), lambda i: (i, 0))],
        out_specs=pl.BlockSpec((HI, 128), lambda i: (0, 0)),
        out_shape=jax.ShapeDtypeStruct((HI, 128), jnp.float32),
    )(dst2d)


_BR = 1024


def _tc_body(relu, agg_ref, deg_ref, w_ref, b_ref, out_ref):
    a = agg_ref[0] + agg_ref[1]
    h = jnp.dot(a, w_ref[...], preferred_element_type=jnp.float32)
    norm = 1.0 / jnp.clip(deg_ref[...], 1.0, None)
    h = h * norm + b_ref[...]
    if relu:
        h = jnp.maximum(h, 0.0)
    out_ref[...] = h


def _tc_layer(agg, deg, w, b, relu):
    return pl.pallas_call(
        functools.partial(_tc_body, relu),
        grid=(N_PAD // _BR,),
        in_specs=[
            pl.BlockSpec((NC, _BR, D), lambda i: (0, i, 0)),
            pl.BlockSpec((_BR, 1), lambda i: (i, 0)),
            pl.BlockSpec((D, D), lambda i: (0, 0)),
            pl.BlockSpec((1, D), lambda i: (0, 0)),
        ],
        out_specs=pl.BlockSpec((_BR, D), lambda i: (i, 0)),
        out_shape=jax.ShapeDtypeStruct((N_PAD, D), jnp.float32),
    )(agg, deg, w, b)


def kernel(x, edge_index, edge_weight, W1, b1, W2, b2):
    src = edge_index[0]
    dst = edge_index[1]
    pad = E_PAD - E
    ar = jnp.arange(pad, dtype=jnp.int32)
    src_p = jnp.concatenate([src, ar % N])
    dst_p = jnp.concatenate([dst, N + ar % (N_PAD - N)])
    w_p = jnp.concatenate([edge_weight, jnp.zeros((pad,), jnp.float32)])
    dst2d = dst_p.reshape(E_PAD // _ER, _ER)
    zrows = jnp.zeros((_CH, D), jnp.float32)

    deg_col = _tc_degree(dst2d).reshape(N_PAD, 1)
    agg1 = _make_sc_agg()(x, src_p, dst_p, w_p, zrows)
    h = _tc_layer(agg1, deg_col, W1, b1.reshape(1, D), relu=True)
    agg2 = _make_sc_agg()(h, src_p, dst_p, w_p, zrows)
    out = _tc_layer(agg2, deg_col, W2, b2.reshape(1, D), relu=False)
    return out[:N]


# speedup vs baseline: 8.5445x; 2.1582x over previous
import functools

import jax
import jax.numpy as jnp
from jax import lax
from jax.experimental import pallas as pl
from jax.experimental.pallas import tpu as pltpu
from jax.experimental.pallas import tpu_sc as plsc

N = 10000
E = 320000
D = 128
NC, NS, L = 2, 16, 16
NW = NC * NS
N_PAD = 10240
HI = N_PAD // 128
RPT = N_PAD // NS
B = 64
NB = 160
E_PAD = NW * B * NB
_CH = 32
_NCH = RPT // _CH
ECB = 8
ECE = ECB * B
NCHK = NB // ECB


@functools.lru_cache(maxsize=None)
def _make_sc_agg():
    mesh = plsc.VectorSubcoreMesh(core_axis_name="c", subcore_axis_name="s",
                                  num_cores=NC, num_subcores=NS)
    out_type = jax.ShapeDtypeStruct((NC, N_PAD, D), jnp.float32)
    scratch = [
        pltpu.VMEM_SHARED((N_PAD, D), jnp.float32),
        pltpu.VMEM((ECB, B), jnp.int32),
        pltpu.VMEM((ECB, B), jnp.int32),
        pltpu.VMEM((ECE,), jnp.float32),
        pltpu.VMEM((B, D), jnp.float32),
        pltpu.VMEM((B, D), jnp.float32),
        pltpu.VMEM((_CH, D), jnp.float32),
        pltpu.SemaphoreType.DMA,
        pltpu.SemaphoreType.DMA,
    ]

    @functools.partial(pl.kernel, out_type=out_type, mesh=mesh,
                       scratch_types=scratch)
    def sc_agg(feat, src2, dst2, wr, zrows, agg_out,
               aggs, srcv, dstv, wv, rows0, rows1, tbuf, semg0, semg1):
        c = lax.axis_index("c")
        s = lax.axis_index("s")
        wid = c * NS + s
        row0 = s * RPT

        pltpu.sync_copy(zrows, tbuf)
        for j in range(_NCH):
            pltpu.sync_copy(tbuf, aggs.at[pl.ds(row0 + j * _CH, _CH)])
        plsc.subcore_barrier()

        def scale(rows, j):
            def g_body(g, _):
                w16 = wv[pl.ds(j * B + g * L, L)]
                for k in range(L):
                    wb = jnp.broadcast_to(w16[k], (L,))
                    e = g * L + k
                    for cc in range(D // L):
                        sl = pl.ds(cc * L, L)
                        rows[e, sl] = rows[e, sl] * wb
                return 0
            lax.fori_loop(0, B // L, g_body, 0)

        def chunk_body(ck, _):
            crow = wid * NB + ck * ECB
            pltpu.sync_copy(src2.at[pl.ds(crow, ECB)], srcv)
            pltpu.sync_copy(dst2.at[pl.ds(crow, ECB)], dstv)
            pltpu.sync_copy(wr.at[pl.ds(crow * B, ECE)], wv)
            pltpu.async_copy(feat.at[srcv.at[0]], rows0, semg0)

            def phase(j, rows, semg, orows, osemg):
                @pl.when(j + 1 < ECB)
                def _():
                    pltpu.async_copy(feat.at[srcv.at[j + 1]], orows, osemg)
                pltpu.make_async_copy(feat.at[srcv.at[j]], rows, semg).wait()
                scale(rows, j)
                pltpu.sync_copy(rows, aggs.at[dstv.at[j]], add=True)

            def pair_body(p, _):
                phase(2 * p, rows0, semg0, rows1, semg1)
                phase(2 * p + 1, rows1, semg1, rows0, semg0)
                return 0

            lax.fori_loop(0, ECB // 2, pair_body, 0)
            return 0

        lax.fori_loop(0, NCHK, chunk_body, 0)
        plsc.subcore_barrier()

        for j in range(_NCH):
            pltpu.sync_copy(aggs.at[pl.ds(row0 + j * _CH, _CH)], tbuf)
            pltpu.sync_copy(tbuf, agg_out.at[c, pl.ds(row0 + j * _CH, _CH)])

    return sc_agg


_ER = 512
_EBR = 8


def _deg_body(dst_ref, deg_ref):
    @pl.when(pl.program_id(0) == 0)
    def _():
        deg_ref[...] = jnp.zeros_like(deg_ref)
    acc = deg_ref[...]
    lanes = lax.broadcasted_iota(jnp.int32, (1, 128), 1)
    his = lax.broadcasted_iota(jnp.int32, (HI, 1), 0)
    for r in range(_EBR):
        d = dst_ref[r, :]
        oh_lo = (d[:, None] % 128 == lanes).astype(jnp.float32)
        oh_hi = (d[None, :] // 128 == his).astype(jnp.float32)
        acc += jnp.dot(oh_hi, oh_lo, preferred_element_type=jnp.float32)
    deg_ref[...] = acc


def _tc_degree(dst2d):
    return pl.pallas_call(
        _deg_body,
        grid=(E_PAD // (_ER * _EBR),),
        in_specs=[pl.BlockSpec((_EBR, _ER), lambda i: (i, 0))],
        out_specs=pl.BlockSpec((HI, 128), lambda i: (0, 0)),
        out_shape=jax.ShapeDtypeStruct((HI, 128), jnp.float32),
    )(dst2d)


_BR = 1024


def _tc_body(relu, agg_ref, deg_ref, w_ref, b_ref, out_ref):
    a = agg_ref[0] + agg_ref[1]
    h = jnp.dot(a, w_ref[...], preferred_element_type=jnp.float32)
    norm = 1.0 / jnp.clip(deg_ref[...], 1.0, None)
    h = h * norm + b_ref[...]
    if relu:
        h = jnp.maximum(h, 0.0)
    out_ref[...] = h


def _tc_layer(agg, deg, w, b, relu):
    return pl.pallas_call(
        functools.partial(_tc_body, relu),
        grid=(N_PAD // _BR,),
        in_specs=[
            pl.BlockSpec((NC, _BR, D), lambda i: (0, i, 0)),
            pl.BlockSpec((_BR, 1), lambda i: (i, 0)),
            pl.BlockSpec((D, D), lambda i: (0, 0)),
            pl.BlockSpec((1, D), lambda i: (0, 0)),
        ],
        out_specs=pl.BlockSpec((_BR, D), lambda i: (i, 0)),
        out_shape=jax.ShapeDtypeStruct((N_PAD, D), jnp.float32),
    )(agg, deg, w, b)


def kernel(x, edge_index, edge_weight, W1, b1, W2, b2):
    src = edge_index[0]
    dst = edge_index[1]
    pad = E_PAD - E
    ar = jnp.arange(pad, dtype=jnp.int32)
    src_p = jnp.concatenate([src, ar % N])
    dst_p = jnp.concatenate([dst, N + ar % (N_PAD - N)])
    w_p = jnp.concatenate([edge_weight, jnp.zeros((pad,), jnp.float32)])
    dst2d = dst_p.reshape(E_PAD // _ER, _ER)
    src2 = src_p.reshape(NW * NB, B)
    dst2 = dst_p.reshape(NW * NB, B)
    zrows = jnp.zeros((_CH, D), jnp.float32)

    deg_col = _tc_degree(dst2d).reshape(N_PAD, 1)
    agg1 = _make_sc_agg()(x, src2, dst2, w_p, zrows)
    h = _tc_layer(agg1, deg_col, W1, b1.reshape(1, D), relu=True)
    agg2 = _make_sc_agg()(h, src2, dst2, w_p, zrows)
    out = _tc_layer(agg2, deg_col, W2, b2.reshape(1, D), relu=False)
    return out[:N]


# speedup vs baseline: 9.2151x; 1.0785x over previous
import functools

import jax
import jax.numpy as jnp
from jax import lax
from jax.experimental import pallas as pl
from jax.experimental.pallas import tpu as pltpu
from jax.experimental.pallas import tpu_sc as plsc

N = 10000
E = 320000
D = 128
NC, NS, L = 2, 16, 16
NW = NC * NS
N_PAD = 10240
HI = N_PAD // 128
RPT = N_PAD // NS
B = 64
NB = 160
E_PAD = NW * B * NB
_CH = 16
_NCH = RPT // _CH
ECB = 8
ECE = ECB * B
NCHK = NB // ECB


@functools.lru_cache(maxsize=None)
def _make_sc_agg():
    mesh = plsc.VectorSubcoreMesh(core_axis_name="c", subcore_axis_name="s",
                                  num_cores=NC, num_subcores=NS)
    out_type = jax.ShapeDtypeStruct((NC, N_PAD, D), jnp.float32)
    scratch = [
        pltpu.VMEM_SHARED((N_PAD, D), jnp.float32),
        [pltpu.VMEM((ECB, B), jnp.int32)] * 2,
        [pltpu.VMEM((ECB, B), jnp.int32)] * 2,
        [pltpu.VMEM((ECE,), jnp.float32)] * 2,
        [pltpu.VMEM((B, D), jnp.float32)] * 3,
        pltpu.VMEM((_CH, D), jnp.float32),
        [pltpu.SemaphoreType.DMA] * 3,
        [pltpu.SemaphoreType.DMA] * 3,
    ]

    @functools.partial(pl.kernel, out_type=out_type, mesh=mesh,
                       scratch_types=scratch)
    def sc_agg(feat, src2, dst2, wr, zrows, agg_out,
               aggs, srcv2, dstv2, wv2, rows3, tbuf, semg3, sems3):
        c = lax.axis_index("c")
        s = lax.axis_index("s")
        wid = c * NS + s
        row0 = s * RPT

        pltpu.sync_copy(zrows, tbuf)
        for j in range(_NCH):
            pltpu.sync_copy(tbuf, aggs.at[pl.ds(row0 + j * _CH, _CH)])
        plsc.subcore_barrier()

        def scale(rows, wv, j):
            def g_body(g, _):
                w16 = wv[pl.ds(j * B + g * L, L)]
                for k in range(L):
                    wb = jnp.broadcast_to(w16[k], (L,))
                    e = g * L + k
                    for cc in range(D // L):
                        sl = pl.ds(cc * L, L)
                        rows[e, sl] = rows[e, sl] * wb
                return 0
            lax.fori_loop(0, B // L, g_body, 0)

        def wait_scat(b, dstv):
            pltpu.make_async_copy(rows3[b], aggs.at[dstv.at[0]],
                                  sems3[b]).wait()

        def chunk_work(ck, srcv, dstv, wv, pdstv, first):
            crow = wid * NB + ck * ECB
            pltpu.sync_copy(src2.at[pl.ds(crow, ECB)], srcv)
            pltpu.sync_copy(dst2.at[pl.ds(crow, ECB)], dstv)
            pltpu.sync_copy(wr.at[pl.ds(crow * B, ECE)], wv)
            if not first:
                wait_scat(0, pdstv)
            pltpu.async_copy(feat.at[srcv.at[0]], rows3[0], semg3[0])
            for j in range(ECB):
                b = j % 3
                if j + 1 < ECB:
                    nb_ = (j + 1) % 3
                    if j < 2:
                        if not first:
                            wait_scat(nb_, pdstv)
                    else:
                        wait_scat(nb_, dstv)
                    pltpu.async_copy(feat.at[srcv.at[j + 1]], rows3[nb_],
                                     semg3[nb_])
                pltpu.make_async_copy(feat.at[srcv.at[j]], rows3[b],
                                      semg3[b]).wait()
                scale(rows3[b], wv, j)
                pltpu.async_copy(rows3[b], aggs.at[dstv.at[j]], sems3[b],
                                 add=True)

        chunk_work(0, srcv2[0], dstv2[0], wv2[0], dstv2[1], True)
        chunk_work(1, srcv2[1], dstv2[1], wv2[1], dstv2[0], False)

        def pair_body(p, _):
            chunk_work(2 + 2 * p, srcv2[0], dstv2[0], wv2[0], dstv2[1], False)
            chunk_work(3 + 2 * p, srcv2[1], dstv2[1], wv2[1], dstv2[0], False)
            return 0

        lax.fori_loop(0, (NCHK - 2) // 2, pair_body, 0)

        for b in range(3):
            wait_scat(b, dstv2[1])
        plsc.subcore_barrier()

        for j in range(_NCH):
            pltpu.sync_copy(aggs.at[pl.ds(row0 + j * _CH, _CH)], tbuf)
            pltpu.sync_copy(tbuf, agg_out.at[c, pl.ds(row0 + j * _CH, _CH)])

    return sc_agg


_ER = 512
_EBR = 8


def _deg_body(dst_ref, deg_ref):
    @pl.when(pl.program_id(0) == 0)
    def _():
        deg_ref[...] = jnp.zeros_like(deg_ref)
    acc = deg_ref[...]
    lanes = lax.broadcasted_iota(jnp.int32, (1, 128), 1)
    his = lax.broadcasted_iota(jnp.int32, (HI, 1), 0)
    for r in range(_EBR):
        d = dst_ref[r, :]
        oh_lo = (d[:, None] % 128 == lanes).astype(jnp.float32)
        oh_hi = (d[None, :] // 128 == his).astype(jnp.float32)
        acc += jnp.dot(oh_hi, oh_lo, preferred_element_type=jnp.float32)
    deg_ref[...] = acc


def _tc_degree(dst2d):
    return pl.pallas_call(
        _deg_body,
        grid=(E_PAD // (_ER * _EBR),),
        in_specs=[pl.BlockSpec((_EBR, _ER), lambda i: (i, 0))],
        out_specs=pl.BlockSpec((HI, 128), lambda i: (0, 0)),
        out_shape=jax.ShapeDtypeStruct((HI, 128), jnp.float32),
    )(dst2d)


_BR = 1024


def _tc_body(relu, agg_ref, deg_ref, w_ref, b_ref, out_ref):
    a = agg_ref[0] + agg_ref[1]
    h = jnp.dot(a, w_ref[...], preferred_element_type=jnp.float32)
    norm = 1.0 / jnp.clip(deg_ref[...], 1.0, None)
    h = h * norm + b_ref[...]
    if relu:
        h = jnp.maximum(h, 0.0)
    out_ref[...] = h


def _tc_layer(agg, deg, w, b, relu):
    return pl.pallas_call(
        functools.partial(_tc_body, relu),
        grid=(N_PAD // _BR,),
        in_specs=[
            pl.BlockSpec((NC, _BR, D), lambda i: (0, i, 0)),
            pl.BlockSpec((_BR, 1), lambda i: (i, 0)),
            pl.BlockSpec((D, D), lambda i: (0, 0)),
            pl.BlockSpec((1, D), lambda i: (0, 0)),
        ],
        out_specs=pl.BlockSpec((_BR, D), lambda i: (i, 0)),
        out_shape=jax.ShapeDtypeStruct((N_PAD, D), jnp.float32),
    )(agg, deg, w, b)


def kernel(x, edge_index, edge_weight, W1, b1, W2, b2):
    src = edge_index[0]
    dst = edge_index[1]
    pad = E_PAD - E
    ar = jnp.arange(pad, dtype=jnp.int32)
    src_p = jnp.concatenate([src, ar % N])
    dst_p = jnp.concatenate([dst, N + ar % (N_PAD - N)])
    w_p = jnp.concatenate([edge_weight, jnp.zeros((pad,), jnp.float32)])
    dst2d = dst_p.reshape(E_PAD // _ER, _ER)
    src2 = src_p.reshape(NW * NB, B)
    dst2 = dst_p.reshape(NW * NB, B)
    zrows = jnp.zeros((_CH, D), jnp.float32)

    deg_col = _tc_degree(dst2d).reshape(N_PAD, 1)
    agg1 = _make_sc_agg()(x, src2, dst2, w_p, zrows)
    h = _tc_layer(agg1, deg_col, W1, b1.reshape(1, D), relu=True)
    agg2 = _make_sc_agg()(h, src2, dst2, w_p, zrows)
    out = _tc_layer(agg2, deg_col, W2, b2.reshape(1, D), relu=False)
    return out[:N]


# speedup vs baseline: 10.1973x; 1.1066x over previous
import functools

import jax
import jax.numpy as jnp
from jax import lax
from jax.experimental import pallas as pl
from jax.experimental.pallas import tpu as pltpu
from jax.experimental.pallas import tpu_sc as plsc

N = 10000
E = 320000
D = 128
NC, NS, L = 2, 16, 16
NW = NC * NS
N_PAD = 10240
HI = N_PAD // 128
RPT = N_PAD // NS
B = 64
NB = 160
E_PAD = NW * B * NB
_CH = 64
_NCH = RPT // _CH
ECB = 8
ECE = ECB * B
NCHK = NB // ECB


@functools.lru_cache(maxsize=None)
def _make_sc_agg():
    mesh = plsc.VectorSubcoreMesh(core_axis_name="c", subcore_axis_name="s",
                                  num_cores=NC, num_subcores=NS)
    out_type = jax.ShapeDtypeStruct((NC, N_PAD, D), jnp.float32)
    scratch = [
        pltpu.VMEM_SHARED((N_PAD, D), jnp.float32),
        pltpu.VMEM((ECB, B), jnp.int32),
        [pltpu.VMEM((ECB, B), jnp.int32)] * 2,
        pltpu.VMEM((ECE,), jnp.float32),
        [pltpu.VMEM((B, D), jnp.float32)] * 3,
        [pltpu.SemaphoreType.DMA] * 3,
        [pltpu.SemaphoreType.DMA] * 3,
    ]

    @functools.partial(pl.kernel, out_type=out_type, mesh=mesh,
                       scratch_types=scratch)
    def sc_agg(feat, src2, dst2, wr, zrows, agg_out,
               aggs, srcv, dstv2, wv, rows3, semg3, sems3):
        c = lax.axis_index("c")
        s = lax.axis_index("s")
        wid = c * NS + s
        row0 = s * RPT

        pltpu.sync_copy(zrows, rows3[0])
        for j in range(_NCH):
            pltpu.async_copy(rows3[0], aggs.at[pl.ds(row0 + j * _CH, _CH)],
                             semg3[0])
        for j in range(_NCH):
            pltpu.make_async_copy(rows3[0],
                                  aggs.at[pl.ds(row0 + j * _CH, _CH)],
                                  semg3[0]).wait()
        plsc.subcore_barrier()

        def scale(rows, j):
            def g_body(g, _):
                w16 = wv[pl.ds(j * B + g * L, L)]
                for k in range(L):
                    wb = jnp.broadcast_to(w16[k], (L,))
                    e = g * L + k
                    for cc in range(D // L):
                        sl = pl.ds(cc * L, L)
                        rows[e, sl] = rows[e, sl] * wb
                return 0
            lax.fori_loop(0, B // L, g_body, 0)

        def wait_scat(b, dstv):
            pltpu.make_async_copy(rows3[b], aggs.at[dstv.at[0]],
                                  sems3[b]).wait()

        def chunk_work(ck, dstv, pdstv, first):
            crow = wid * NB + ck * ECB
            pltpu.sync_copy(src2.at[pl.ds(crow, ECB)], srcv)
            pltpu.sync_copy(dst2.at[pl.ds(crow, ECB)], dstv)
            pltpu.sync_copy(wr.at[pl.ds(crow * B, ECE)], wv)
            if not first:
                wait_scat(0, pdstv)
            pltpu.async_copy(feat.at[srcv.at[0]], rows3[0], semg3[0])
            for j in range(ECB):
                b = j % 3
                if j + 1 < ECB:
                    nb_ = (j + 1) % 3
                    if j < 2:
                        if not first:
                            wait_scat(nb_, pdstv)
                    else:
                        wait_scat(nb_, dstv)
                    pltpu.async_copy(feat.at[srcv.at[j + 1]], rows3[nb_],
                                     semg3[nb_])
                pltpu.make_async_copy(feat.at[srcv.at[j]], rows3[b],
                                      semg3[b]).wait()
                scale(rows3[b], j)
                pltpu.async_copy(rows3[b], aggs.at[dstv.at[j]], sems3[b],
                                 add=True)

        chunk_work(0, dstv2[0], dstv2[1], True)
        chunk_work(1, dstv2[1], dstv2[0], False)

        def pair_body(p, _):
            chunk_work(2 + 2 * p, dstv2[0], dstv2[1], False)
            chunk_work(3 + 2 * p, dstv2[1], dstv2[0], False)
            return 0

        lax.fori_loop(0, (NCHK - 2) // 2, pair_body, 0)

        for b in range(3):
            wait_scat(b, dstv2[1])
        plsc.subcore_barrier()

        for j in range(_NCH):
            b = j % 3
            if j >= 3:
                pltpu.make_async_copy(
                    rows3[b], agg_out.at[c, pl.ds(row0 + (j - 3) * _CH, _CH)],
                    sems3[b]).wait()
            pltpu.sync_copy(aggs.at[pl.ds(row0 + j * _CH, _CH)], rows3[b])
            pltpu.async_copy(rows3[b],
                             agg_out.at[c, pl.ds(row0 + j * _CH, _CH)],
                             sems3[b])
        for j in range(_NCH - 3, _NCH):
            pltpu.make_async_copy(rows3[j % 3],
                                  agg_out.at[c, pl.ds(row0 + j * _CH, _CH)],
                                  sems3[j % 3]).wait()

    return sc_agg


_ER = 512
_EBR = 8


def _deg_body(dst_ref, deg_ref):
    @pl.when(pl.program_id(0) == 0)
    def _():
        deg_ref[...] = jnp.zeros_like(deg_ref)
    acc = deg_ref[...]
    lanes = lax.broadcasted_iota(jnp.int32, (1, 128), 1)
    his = lax.broadcasted_iota(jnp.int32, (HI, 1), 0)
    for r in range(_EBR):
        d = dst_ref[r, :]
        oh_lo = (d[:, None] % 128 == lanes).astype(jnp.float32)
        oh_hi = (d[None, :] // 128 == his).astype(jnp.float32)
        acc += jnp.dot(oh_hi, oh_lo, preferred_element_type=jnp.float32)
    deg_ref[...] = acc


def _tc_degree(dst2d):
    return pl.pallas_call(
        _deg_body,
        grid=(E_PAD // (_ER * _EBR),),
        in_specs=[pl.BlockSpec((_EBR, _ER), lambda i: (i, 0))],
        out_specs=pl.BlockSpec((HI, 128), lambda i: (0, 0)),
        out_shape=jax.ShapeDtypeStruct((HI, 128), jnp.float32),
    )(dst2d)


_BR = 1024


def _tc_body(relu, agg_ref, deg_ref, w_ref, b_ref, out_ref):
    a = agg_ref[0] + agg_ref[1]
    h = jnp.dot(a, w_ref[...], preferred_element_type=jnp.float32)
    norm = 1.0 / jnp.clip(deg_ref[...], 1.0, None)
    h = h * norm + b_ref[...]
    if relu:
        h = jnp.maximum(h, 0.0)
    out_ref[...] = h


def _tc_layer(agg, deg, w, b, relu, n_out):
    return pl.pallas_call(
        functools.partial(_tc_body, relu),
        grid=(N_PAD // _BR,),
        in_specs=[
            pl.BlockSpec((NC, _BR, D), lambda i: (0, i, 0)),
            pl.BlockSpec((_BR, 1), lambda i: (i, 0)),
            pl.BlockSpec((D, D), lambda i: (0, 0)),
            pl.BlockSpec((1, D), lambda i: (0, 0)),
        ],
        out_specs=pl.BlockSpec((_BR, D), lambda i: (i, 0)),
        out_shape=jax.ShapeDtypeStruct((n_out, D), jnp.float32),
    )(agg, deg, w, b)


def kernel(x, edge_index, edge_weight, W1, b1, W2, b2):
    src = edge_index[0]
    dst = edge_index[1]
    pad = E_PAD - E
    ar = jnp.arange(pad, dtype=jnp.int32)
    src_p = jnp.concatenate([src, ar % N])
    dst_p = jnp.concatenate([dst, N + ar % (N_PAD - N)])
    w_p = jnp.concatenate([edge_weight, jnp.zeros((pad,), jnp.float32)])
    dst2d = dst_p.reshape(E_PAD // _ER, _ER)
    src2 = src_p.reshape(NW * NB, B)
    dst2 = dst_p.reshape(NW * NB, B)
    zrows = jnp.zeros((_CH, D), jnp.float32)

    agg1 = _make_sc_agg()(x, src2, dst2, w_p, zrows)
    deg_col = _tc_degree(dst2d).reshape(N_PAD, 1)
    h = _tc_layer(agg1, deg_col, W1, b1.reshape(1, D), relu=True, n_out=N_PAD)
    agg2 = _make_sc_agg()(h, src2, dst2, w_p, zrows)
    out = _tc_layer(agg2, deg_col, W2, b2.reshape(1, D), relu=False, n_out=N)
    return out
